# baseline (speedup 1.0000x reference)
import functools

import jax
import jax.numpy as jnp
from jax import lax
from jax.experimental import pallas as pl
from jax.experimental.pallas import tpu as pltpu
from jax.experimental.pallas import tpu_sc as plsc

NUM_CORES = 2
NUM_SUBCORES = 16
NUM_WORKERS = NUM_CORES * NUM_SUBCORES
LANES = 16

HIDDEN = 128
CHUNK = 128
NBUF = 4


def _rsqrt(x):
    i = lax.bitcast_convert_type(x, jnp.int32)
    i = jnp.int32(0x5F3759DF) - lax.shift_right_logical(i, 1)
    y = lax.bitcast_convert_type(i, jnp.float32)
    for _ in range(3):
        y = y * (jnp.float32(1.5) - jnp.float32(0.5) * x * y * y)
    return y


def _sc_body(ids_hbm, tt_hbm, word_hbm, pos_hbm, dt_hbm, gamma_hbm,
             beta_hbm, out_hbm, pos_sh, dt_v, gamma_v, beta_v, idx_v,
             tt_v, rows_v, ids_sem, tts_sem, f_sem, g_sem, o_sem):
    wid = lax.axis_index("s") * NUM_CORES + lax.axis_index("c")
    n_tokens = ids_hbm.shape[0]
    per_w = n_tokens // NUM_WORKERS
    n_chunks = per_w // CHUNK
    wstart = wid * per_w

    def chunk_base(g):
        return wstart + g * CHUNK

    @pl.when(lax.axis_index("s") == 0)
    def _():
        pltpu.sync_copy(pos_hbm, pos_sh)

    pltpu.sync_copy(dt_hbm, dt_v)
    pltpu.sync_copy(gamma_hbm, gamma_v)
    pltpu.sync_copy(beta_hbm, beta_v)
    plsc.subcore_barrier()

    nvec = HIDDEN // LANES
    inv_h = jnp.float32(1.0 / HIDDEN)

    def ids_dma(g, b):
        base = chunk_base(g)
        return pltpu.make_async_copy(
            ids_hbm.at[pl.ds(base, CHUNK)], idx_v.at[b], ids_sem.at[b])

    def tts_dma(g, b):
        base = chunk_base(g)
        return pltpu.make_async_copy(
            tt_hbm.at[pl.ds(base, CHUNK)],
            tt_v.at[b, pl.ds(0, CHUNK)], tts_sem.at[b])

    def fill_dma(g, b):
        pos_base = (g % (512 // CHUNK)) * CHUNK
        return pltpu.make_async_copy(
            pos_sh.at[pl.ds(pos_base, CHUNK)], rows_v.at[b], f_sem.at[b])

    def gather_dma(b):
        return pltpu.make_async_copy(
            word_hbm.at[idx_v.at[b]], rows_v.at[b], g_sem.at[b])

    def out_dma(g, b):
        base = chunk_base(g)
        return pltpu.make_async_copy(
            rows_v.at[b], out_hbm.at[pl.ds(base, CHUNK)], o_sem.at[b])

    def compute(b):
        gammas = [gamma_v[pl.ds(j * LANES, LANES)] for j in range(nvec)]
        betas = [beta_v[pl.ds(j * LANES, LANES)] for j in range(nvec)]
        dts = [dt_v[pl.ds(j * LANES, LANES)] for j in range(nvec)]

        @plsc.parallel_loop(0, CHUNK, unroll=3)
        def _tokens(t):
            tt = tt_v[b, pl.ds(t, LANES)][0]
            ttf = jnp.full((LANES,), lax.convert_element_type(
                tt, jnp.float32))
            xs = []
            acc = None
            vacc = None
            for j in range(nvec):
                sl = pl.ds(j * LANES, LANES)
                x = rows_v[b, t, sl] + ttf * dts[j]
                xs.append(x)
                acc = x if acc is None else acc + x
                sq = x * x
                vacc = sq if vacc is None else vacc + sq
            mean = jnp.sum(acc) * inv_h
            var = jnp.sum(vacc) * inv_h - mean * mean
            inv = _rsqrt(var + jnp.float32(1e-12))
            inv_s = jnp.full((LANES,), inv)
            mi_s = jnp.full((LANES,), mean * inv)
            for j in range(nvec):
                sl = pl.ds(j * LANES, LANES)
                rows_v[b, t, sl] = (
                    (xs[j] * inv_s - mi_s) * gammas[j] + betas[j])

    ids_dma(0, 0).start()
    tts_dma(0, 0).start()
    ids_dma(1, 1).start()
    tts_dma(1, 1).start()
    fill_dma(0, 0).start()
    fill_dma(1, 1).start()
    fill_dma(0, 0).wait()
    ids_dma(0, 0).wait()
    gather_dma(0).start(add=True)

    @pl.loop(0, n_chunks, step=NBUF)
    def _chunks(g0):
        for b in range(NBUF):
            g = g0 + b
            nb = (b + 1) % NBUF
            fb = (b + 2) % NBUF

            @pl.when(g + 1 < n_chunks)
            def _():
                fill_dma(g + 1, nb).wait()
                ids_dma(g + 1, nb).wait()
                gather_dma(nb).start(add=True)

            gather_dma(b).wait()

            @pl.when(g + 2 < n_chunks)
            def _():
                ids_dma(g + 2, fb).start()
                tts_dma(g + 2, fb).start()

                @pl.when(g >= 2)
                def _():
                    out_dma(g - 2, fb).wait()

                fill_dma(g + 2, fb).start()

            tts_dma(g, b).wait()
            compute(b)
            out_dma(g, b).start()

    out_dma(n_chunks - 2, (n_chunks - 2) % NBUF).wait()
    out_dma(n_chunks - 1, (n_chunks - 1) % NBUF).wait()


def kernel(input_ids, token_type_ids, word_emb, pos_emb, type_emb, gamma,
           beta):
    b, s = input_ids.shape
    n = b * s
    ids_flat = input_ids.reshape(n).astype(jnp.int32)
    tt_flat = token_type_ids.reshape(n).astype(jnp.int32)
    pos_fused = pos_emb + type_emb[0][None, :]
    dt = type_emb[1] - type_emb[0]

    mesh = plsc.VectorSubcoreMesh(core_axis_name="c", subcore_axis_name="s")
    out_flat = pl.kernel(
        _sc_body,
        out_type=jax.ShapeDtypeStruct((n, HIDDEN), jnp.float32),
        mesh=mesh,
        compiler_params=pltpu.CompilerParams(needs_layout_passes=False),
        scratch_types=[
            pltpu.VMEM_SHARED((512, HIDDEN), jnp.float32),
            pltpu.VMEM((HIDDEN,), jnp.float32),
            pltpu.VMEM((HIDDEN,), jnp.float32),
            pltpu.VMEM((HIDDEN,), jnp.float32),
            pltpu.VMEM((NBUF, CHUNK), jnp.int32),
            pltpu.VMEM((NBUF, CHUNK + LANES), jnp.int32),
            pltpu.VMEM((NBUF, CHUNK, HIDDEN), jnp.float32),
            pltpu.SemaphoreType.DMA((NBUF,)),
            pltpu.SemaphoreType.DMA((NBUF,)),
            pltpu.SemaphoreType.DMA((NBUF,)),
            pltpu.SemaphoreType.DMA((NBUF,)),
            pltpu.SemaphoreType.DMA((NBUF,)),
        ],
    )(ids_flat, tt_flat, word_emb, pos_fused, dt, gamma, beta)
    return out_flat.reshape(b, s, HIDDEN)


# speedup vs baseline: 1.1219x; 1.1219x over previous
import functools

import jax
import jax.numpy as jnp
from jax import lax
from jax.experimental import pallas as pl
from jax.experimental.pallas import tpu as pltpu
from jax.experimental.pallas import tpu_sc as plsc

NUM_CORES = 2
NUM_SUBCORES = 16
NUM_WORKERS = NUM_CORES * NUM_SUBCORES
LANES = 16

HIDDEN = 128
CHUNK = 128
NBUF = 4


def _rsqrt(x):
    i = lax.bitcast_convert_type(x, jnp.int32)
    i = jnp.int32(0x5F3759DF) - lax.shift_right_logical(i, 1)
    y = lax.bitcast_convert_type(i, jnp.float32)
    for _ in range(3):
        y = y * (jnp.float32(1.5) - jnp.float32(0.5) * x * y * y)
    return y


def _sc_body(ids_hbm, tt_hbm, word_hbm, pos_hbm, type_hbm, gamma_hbm,
             beta_hbm, out_hbm, pos_sh, type_v, gamma_v, beta_v, idx_v,
             tt_v, rows_v, ids_sem, tts_sem, f_sem, g_sem, o_sem):
    wid = lax.axis_index("s") * NUM_CORES + lax.axis_index("c")
    n_tokens = ids_hbm.shape[0]
    per_w = n_tokens // NUM_WORKERS
    n_chunks = per_w // CHUNK
    wstart = wid * per_w

    def chunk_base(g):
        return wstart + g * CHUNK

    @pl.when(lax.axis_index("s") == 0)
    def _():
        pltpu.sync_copy(pos_hbm, pos_sh)

    pltpu.sync_copy(type_hbm, type_v)
    pltpu.sync_copy(gamma_hbm, gamma_v)
    pltpu.sync_copy(beta_hbm, beta_v)
    plsc.subcore_barrier()

    nvec = HIDDEN // LANES
    inv_h = jnp.float32(1.0 / HIDDEN)

    def ids_dma(g, b):
        base = chunk_base(g)
        return pltpu.make_async_copy(
            ids_hbm.at[pl.ds(base, CHUNK)], idx_v.at[b], ids_sem.at[b])

    def tts_dma(g, b):
        base = chunk_base(g)
        return pltpu.make_async_copy(
            tt_hbm.at[pl.ds(base, CHUNK)],
            tt_v.at[b, pl.ds(0, CHUNK)], tts_sem.at[b])

    def fill_dma(g, b):
        pos_base = (g % (512 // CHUNK)) * CHUNK
        return pltpu.make_async_copy(
            pos_sh.at[pl.ds(pos_base, CHUNK)], rows_v.at[b], f_sem.at[b])

    def gather_dma(b):
        return pltpu.make_async_copy(
            word_hbm.at[idx_v.at[b]], rows_v.at[b], g_sem.at[b])

    def out_dma(g, b):
        base = chunk_base(g)
        return pltpu.make_async_copy(
            rows_v.at[b], out_hbm.at[pl.ds(base, CHUNK)], o_sem.at[b])

    def compute(b):
        gammas = [gamma_v[pl.ds(j * LANES, LANES)] for j in range(nvec)]
        betas = [beta_v[pl.ds(j * LANES, LANES)] for j in range(nvec)]

        @plsc.parallel_loop(0, CHUNK, unroll=4)
        def _tokens(t):
            tt = tt_v[b, pl.ds(t, LANES)][0]
            xs = []
            acc = None
            vacc = None
            for j in range(nvec):
                sl = pl.ds(j * LANES, LANES)
                x = rows_v[b, t, sl] + type_v[tt, sl]
                xs.append(x)
                acc = x if acc is None else acc + x
                sq = x * x
                vacc = sq if vacc is None else vacc + sq
            mean = jnp.sum(acc) * inv_h
            var = jnp.sum(vacc) * inv_h - mean * mean
            inv = _rsqrt(var + jnp.float32(1e-12))
            inv_s = jnp.full((LANES,), inv)
            mi_s = jnp.full((LANES,), mean * inv)
            for j in range(nvec):
                sl = pl.ds(j * LANES, LANES)
                rows_v[b, t, sl] = (
                    (xs[j] * inv_s - mi_s) * gammas[j] + betas[j])

    ids_dma(0, 0).start()
    tts_dma(0, 0).start()
    ids_dma(1, 1).start()
    tts_dma(1, 1).start()
    fill_dma(0, 0).start()
    fill_dma(1, 1).start()
    fill_dma(0, 0).wait()
    ids_dma(0, 0).wait()
    gather_dma(0).start(add=True)

    @pl.loop(0, n_chunks, step=NBUF)
    def _chunks(g0):
        for b in range(NBUF):
            g = g0 + b
            nb = (b + 1) % NBUF
            fb = (b + 2) % NBUF

            @pl.when(g + 1 < n_chunks)
            def _():
                fill_dma(g + 1, nb).wait()
                ids_dma(g + 1, nb).wait()
                gather_dma(nb).start(add=True)

            gather_dma(b).wait()

            @pl.when(g + 2 < n_chunks)
            def _():
                ids_dma(g + 2, fb).start()
                tts_dma(g + 2, fb).start()

                @pl.when(g >= 2)
                def _():
                    out_dma(g - 2, fb).wait()

                fill_dma(g + 2, fb).start()

            tts_dma(g, b).wait()
            compute(b)
            out_dma(g, b).start()

    out_dma(n_chunks - 2, (n_chunks - 2) % NBUF).wait()
    out_dma(n_chunks - 1, (n_chunks - 1) % NBUF).wait()


def kernel(input_ids, token_type_ids, word_emb, pos_emb, type_emb, gamma,
           beta):
    b, s = input_ids.shape
    n = b * s
    ids_flat = input_ids.reshape(n).astype(jnp.int32)
    tt_flat = token_type_ids.reshape(n).astype(jnp.int32)

    mesh = plsc.VectorSubcoreMesh(core_axis_name="c", subcore_axis_name="s")
    out_flat = pl.kernel(
        _sc_body,
        out_type=jax.ShapeDtypeStruct((n, HIDDEN), jnp.float32),
        mesh=mesh,
        compiler_params=pltpu.CompilerParams(needs_layout_passes=False),
        scratch_types=[
            pltpu.VMEM_SHARED((512, HIDDEN), jnp.float32),
            pltpu.VMEM((2, HIDDEN), jnp.float32),
            pltpu.VMEM((HIDDEN,), jnp.float32),
            pltpu.VMEM((HIDDEN,), jnp.float32),
            pltpu.VMEM((NBUF, CHUNK), jnp.int32),
            pltpu.VMEM((NBUF, CHUNK + LANES), jnp.int32),
            pltpu.VMEM((NBUF, CHUNK, HIDDEN), jnp.float32),
            pltpu.SemaphoreType.DMA((NBUF,)),
            pltpu.SemaphoreType.DMA((NBUF,)),
            pltpu.SemaphoreType.DMA((NBUF,)),
            pltpu.SemaphoreType.DMA((NBUF,)),
            pltpu.SemaphoreType.DMA((NBUF,)),
        ],
    )(ids_flat, tt_flat, word_emb, pos_emb, type_emb, gamma, beta)
    return out_flat.reshape(b, s, HIDDEN)
